# baseline (speedup 1.0000x reference)
import functools

import jax
import jax.numpy as jnp
from jax import lax
from jax.experimental import pallas as pl
from jax.experimental.pallas import tpu as pltpu
from jax.experimental.pallas import tpu_sc as plsc

DIM = 16
BATCH = 4096
RPG = 128 // DIM
NC = 2
NS = 16
NW = NC * NS
UPW = BATCH // NW
IPS = BATCH // NS
ICH = IPS // 128


@functools.partial(
    pl.kernel,
    mesh=plsc.VectorSubcoreMesh(core_axis_name="c", subcore_axis_name="s"),
    out_type=jax.ShapeDtypeStruct((BATCH,), jnp.float32),
    compiler_params=pltpu.CompilerParams(needs_layout_passes=False),
    scratch_types=[
        pltpu.VMEM((UPW,), jnp.int32),
        pltpu.VMEM((UPW,), jnp.int32),
        pltpu.VMEM((ICH, 128), jnp.int32),
        pltpu.VMEM((ICH, 128), jnp.int32),
        pltpu.VMEM((UPW, 128), jnp.float32),
        pltpu.VMEM((ICH, 128, 128), jnp.float32),
        pltpu.VMEM((128,), jnp.float32),
        pltpu.VMEM((NS, 128), jnp.float32),
        pltpu.VMEM((UPW,), jnp.float32),
        pltpu.VMEM((16, 16), jnp.float32),
        pltpu.VMEM_SHARED((NS, 128), jnp.float32),
        pltpu.SemaphoreType.DMA,
        pltpu.SemaphoreType.DMA,
    ],
)
def _mf_kernel(users_hbm, items_hbm, ug_hbm, ig_hbm, u_hbm, v_hbm, out_hbm,
               uidx, ugidx, iidx, igidx, urows, vrows, part, allparts,
               outv, tile, shared, sem_u, sem_i):
    c = lax.axis_index("c")
    s = lax.axis_index("s")
    wid = s * NC + c
    ubase = wid * UPW
    ibase = s * IPS
    lane = lax.iota(jnp.int32, 16)

    pltpu.sync_copy(users_hbm.at[pl.ds(ubase, UPW)], uidx)
    pltpu.sync_copy(ug_hbm.at[pl.ds(ubase, UPW)], ugidx)
    for j in range(ICH):
        pltpu.sync_copy(items_hbm.at[pl.ds(ibase + j * 128, 128)], iidx.at[j])
        pltpu.sync_copy(ig_hbm.at[pl.ds(ibase + j * 128, 128)], igidx.at[j])

    ucp = pltpu.async_copy(u_hbm.at[ugidx], urows, sem_u)
    icps = [pltpu.async_copy(v_hbm.at[igidx.at[j]], vrows.at[j], sem_i)
            for j in range(ICH)]
    for cp in icps:
        cp.wait()

    def isum_body(g, accs):
        j = g // 8
        g2 = g - j * 8
        rowv = g2 * 16 + lane
        colbase = (iidx[j, pl.ds(g2 * 16, 16)] & (RPG - 1)) * DIM
        jv = jnp.full((16,), j, jnp.int32)
        return tuple(
            accs[d] + plsc.load_gather(vrows, [jv, rowv, colbase + d])
            for d in range(DIM))
    accs = lax.fori_loop(
        0, IPS // 16, isum_body,
        tuple(jnp.zeros((16,), jnp.float32) for _ in range(DIM)))

    for d in range(DIM):
        plsc.store_scatter(tile, [lane, jnp.full((16,), d, jnp.int32)],
                           accs[d])
    acc = tile[0, :]
    for r in range(1, 16):
        acc = acc + tile[r, :]
    part[pl.ds(0, DIM)] = acc

    pltpu.sync_copy(part, shared.at[s])
    plsc.subcore_barrier()
    pltpu.sync_copy(shared, allparts)
    svec = jnp.zeros((DIM,), jnp.float32)
    for t in range(NS):
        svec = svec + allparts[t, pl.ds(0, DIM)]

    sd = [jnp.take_along_axis(svec, jnp.full((16,), d, jnp.int32), axis=0)
          for d in range(DIM)]

    ucp.wait()

    def dot_body(g, carry):
        rowv = g * 16 + lane
        colbase = (uidx[pl.ds(g * 16, 16)] & (RPG - 1)) * DIM
        ovec = jnp.zeros((16,), jnp.float32)
        for d in range(DIM):
            ovec = ovec + sd[d] * plsc.load_gather(urows, [rowv, colbase + d])
        outv[pl.ds(g * 16, 16)] = ovec
        return carry
    lax.fori_loop(0, UPW // 16, dot_body, jnp.int32(0))

    pltpu.sync_copy(outv, out_hbm.at[pl.ds(ubase, UPW)])


def kernel(users, items, U, V):
    users = users.astype(jnp.int32)
    items = items.astype(jnp.int32)
    u2 = U.reshape(U.shape[0] // RPG, 128)
    v2 = V.reshape(V.shape[0] // RPG, 128)
    return _mf_kernel(users, items,
                      lax.shift_right_logical(users, 3),
                      lax.shift_right_logical(items, 3),
                      u2, v2)
